# baseline (speedup 1.0000x reference)
import jax
import jax.numpy as jnp
from jax.experimental import pallas as pl
from jax.experimental.pallas import tpu as pltpu

_B, _N = 4, 1024
_TD = 512


def _fused_body(v5_ref, x1_ref, v1_ref, v7r_ref, x10_ref, x11_ref, x12_ref,
                x6_scr):
    k = pl.program_id(1)

    @pl.when(k == 0)
    def _colmax():
        x6_scr[0, :] = jnp.max(v5_ref[0], axis=0)

    @pl.when(k > 0)
    def _main():
        x1t = x1_ref[0]
        v1t = v1_ref[0]
        v7t = v7r_ref[0]
        x6v = x6_scr[0]
        x9 = jax.nn.sigmoid(x1t + v7t)
        p = x9 * v1t
        top_a = p * x1t
        x6col = x6v[:, None]
        top_b = p * x6col
        x6b = jnp.broadcast_to(x6col, x1t.shape)
        x10_ref[0, :_N, :] = x1t
        x10_ref[0, _N:, :] = x6b
        x12_ref[0, :_N, :] = x1t + top_a
        x12_ref[0, _N:, :] = x6b + top_b
        x11_ref[0, :, :_N] = top_a.T
        x11_ref[0, :, _N:] = top_b.T


def kernel(x1, v1, v5, v6r, v7r):
    del v6r
    B, N, D = x1.shape

    def _strip(b, k):
        kk = jnp.maximum(k - 1, 0)
        return (b, 0, kk)

    x10, x11, x12 = pl.pallas_call(
        _fused_body,
        grid=(B, 1 + D // _TD),
        in_specs=[
            pl.BlockSpec((1, N, D), lambda b, k: (b, 0, 0)),
            pl.BlockSpec((1, N, _TD), _strip),
            pl.BlockSpec((1, N, _TD), _strip),
            pl.BlockSpec((1, N, _TD), _strip),
        ],
        out_specs=[
            pl.BlockSpec((1, 2 * N, _TD), _strip),
            pl.BlockSpec((1, _TD, 2 * N),
                         lambda b, k: (b, jnp.maximum(k - 1, 0), 0)),
            pl.BlockSpec((1, 2 * N, _TD), _strip),
        ],
        out_shape=[
            jax.ShapeDtypeStruct((B, 2 * N, D), jnp.float32),
            jax.ShapeDtypeStruct((B, D, 2 * N), jnp.float32),
            jax.ShapeDtypeStruct((B, 2 * N, D), jnp.float32),
        ],
        scratch_shapes=[pltpu.VMEM((1, N), jnp.float32)],
    )(v5, x1, v1, v7r)
    return (x10, x11, x12)
